# baseline (speedup 1.0000x reference)
import functools

import jax
import jax.numpy as jnp
from jax import lax
from jax.experimental import pallas as pl
from jax.experimental.pallas import tpu as pltpu
from jax.experimental.pallas import tpu_sc as plsc

_N = 10000
_E = 320000
_D = 128
_NW = 32
_EPW = _E // _NW
_C = 80
_NCH = _EPW // _C
_G = _C // 16
_R = 3


def _sq_rows_to_tbuf(src, tbuf_v):
    @plsc.parallel_loop(0, _C, step=1, unroll=4)
    def e_body(e):
        p = [src[e, pl.ds(k * 16, 16)] for k in range(_D // 16)]
        p = [x * x for x in p]
        while len(p) > 1:
            p = [p[i] + p[i + 1] for i in range(0, len(p), 2)]
        tbuf_v[pl.ds(e * 16, 16)] = p[0]


def _transpose_sum(tbuf_v, g):
    ids = lax.iota(jnp.int32, 16) * 16 + g * 256
    o = jnp.zeros((16,), jnp.float32)
    for l in range(16):
        o = o + plsc.load_gather(tbuf_v, [ids + l])
    return o


@functools.partial(
    pl.kernel,
    mesh=plsc.VectorSubcoreMesh(core_axis_name="c", subcore_axis_name="s"),
    out_type=jax.ShapeDtypeStruct((_E,), jnp.float32),
    compiler_params=pltpu.CompilerParams(needs_layout_passes=False),
    scratch_types=[
        pltpu.VMEM_SHARED((_N, _D), jnp.float32),
        pltpu.VMEM_SHARED((_N,), jnp.float32),
        [pltpu.VMEM((_C,), jnp.int32) for _ in range(_R)],
        [pltpu.VMEM((_C,), jnp.int32) for _ in range(_R)],
        [pltpu.VMEM((_C, _D), jnp.float32) for _ in range(_R)],
        pltpu.VMEM((_N,), jnp.float32),
        [pltpu.VMEM((_C,), jnp.float32) for _ in range(_R)],
        pltpu.VMEM((_C * 16,), jnp.float32),
        [pltpu.SemaphoreType.DMA for _ in range(_R)],
        [pltpu.SemaphoreType.DMA for _ in range(_R)],
        [pltpu.SemaphoreType.DMA for _ in range(_R)],
        [pltpu.SemaphoreType.DMA for _ in range(_R)],
    ],
)
def _edge_dot(d0_hbm, d1_hbm, table_hbm, out_hbm,
              table_sh, norms_sh, idx0_bufs, idx1_bufs, cbufs,
              norms_v, out_bufs, tbuf_v,
              isems, asems, bsems, osems):
    cid = lax.axis_index("c")
    sid = lax.axis_index("s")
    wid = sid * 2 + cid
    base_w = wid * _EPW

    rows_per_sub = 624
    pltpu.sync_copy(table_hbm.at[pl.ds(sid * rows_per_sub, rows_per_sub)],
                    table_sh.at[pl.ds(sid * rows_per_sub, rows_per_sub)])

    @pl.when(sid == 0)
    def _():
        rem = 16 * rows_per_sub
        pltpu.sync_copy(table_hbm.at[pl.ds(rem, _N - rem)],
                        table_sh.at[pl.ds(rem, _N - rem)])

    def _norm_blocks(node_base, nblocks):
        def blk_body(blk, carry):
            off = node_base + blk * _C
            pltpu.sync_copy(table_hbm.at[pl.ds(off, _C)], cbufs[0])
            _sq_rows_to_tbuf(cbufs[0], tbuf_v)

            @plsc.parallel_loop(0, _G, step=1, unroll=1)
            def g_body(g):
                out_bufs[0][pl.ds(g * 16, 16)] = _transpose_sum(tbuf_v, g)

            pltpu.sync_copy(out_bufs[0], norms_sh.at[pl.ds(off, _C)])
            return carry

        lax.fori_loop(0, nblocks, blk_body, 0)

    @pl.when(sid < 15)
    def _():
        _norm_blocks(sid * 640, 8)

    @pl.when(sid == 15)
    def _():
        _norm_blocks(9600, 5)

    plsc.subcore_barrier()

    pltpu.sync_copy(norms_sh, norms_v)

    def issue_idx(t, b):
        off = base_w + t * _C
        pltpu.async_copy(d0_hbm.at[pl.ds(off, _C)], idx0_bufs[b], isems[b])
        pltpu.async_copy(d1_hbm.at[pl.ds(off, _C)], idx1_bufs[b], isems[b])

    def wait_idx(b):
        pltpu.make_async_copy(d0_hbm.at[pl.ds(0, _C)], idx0_bufs[b],
                              isems[b]).wait()
        pltpu.make_async_copy(d1_hbm.at[pl.ds(0, _C)], idx1_bufs[b],
                              isems[b]).wait()

    def issue_a(b):
        pltpu.async_copy(table_sh.at[idx0_bufs[b]], cbufs[b], asems[b])

    def wait_a(b):
        pltpu.make_async_copy(table_hbm.at[pl.ds(0, _C)], cbufs[b],
                              asems[b]).wait()

    def issue_b(b):
        pltpu.async_copy(table_sh.at[idx1_bufs[b]], cbufs[b], bsems[b],
                         add=True)

    def wait_b(b):
        pltpu.make_async_copy(table_hbm.at[pl.ds(0, _C)], cbufs[b],
                              bsems[b]).wait()

    def drain_out(b):
        pltpu.make_async_copy(out_bufs[b], out_hbm.at[pl.ds(0, _C)],
                              osems[b]).wait()

    def compute_chunk(b):
        cb = cbufs[b]
        ob = out_bufs[b]
        _sq_rows_to_tbuf(cb, tbuf_v)

        @plsc.parallel_loop(0, _G, step=1, unroll=1)
        def group_body(g):
            o = _transpose_sum(tbuf_v, g)
            n0 = plsc.load_gather(norms_v, [idx0_bufs[b][pl.ds(g * 16, 16)]])
            n1 = plsc.load_gather(norms_v, [idx1_bufs[b][pl.ds(g * 16, 16)]])
            half = jnp.full((16,), 0.5, jnp.float32)
            ob[pl.ds(g * 16, 16)] = (o - n0 - n1) * half

    def issue_out(t, b):
        pltpu.async_copy(out_bufs[b],
                         out_hbm.at[pl.ds(base_w + t * _C, _C)], osems[b])

    def step(t, b, tail):
        wait_b(b)

        if tail is None or tail + 1 < _NCH:
            @pl.when(t + 1 < _NCH)
            def _():
                wait_a((b + 1) % _R)
                issue_b((b + 1) % _R)

        if tail is None or tail >= _R:
            @pl.when(t >= _R)
            def _():
                drain_out(b)

        compute_chunk(b)
        issue_out(t, b)

        if tail is None:
            @pl.when(t + _R < _NCH)
            def _():
                issue_idx(t + _R, b)

            @pl.when(t + 2 < _NCH)
            def _():
                wait_idx((b + 2) % _R)
                issue_a((b + 2) % _R)

    for c in range(_R):
        issue_idx(c, c)
    for c in range(2):
        wait_idx(c)
        issue_a(c)
    wait_a(0)
    issue_b(0)

    def loop_body(tt, carry):
        for b in range(_R):
            step(tt * _R + b, b, None)
        return carry

    n_main = (_NCH // _R) * _R
    lax.fori_loop(0, _NCH // _R, loop_body, 0)

    for t in range(n_main, _NCH):
        step(t, t % _R, t)

    for b in range(_R):
        drain_out(b)


def kernel(data, embedding):
    return _edge_dot(data[0], data[1], embedding)


# speedup vs baseline: 1.3775x; 1.3775x over previous
import functools

import jax
import jax.numpy as jnp
from jax import lax
from jax.experimental import pallas as pl
from jax.experimental.pallas import tpu as pltpu
from jax.experimental.pallas import tpu_sc as plsc

_N = 10000
_E = 320000
_D = 128
_NW = 32
_EPW = _E // _NW
_C = 80
_NCH = _EPW // _C
_G = _C // 16


@functools.partial(
    pl.kernel,
    mesh=plsc.VectorSubcoreMesh(core_axis_name="c", subcore_axis_name="s"),
    out_type=jax.ShapeDtypeStruct((_E,), jnp.float32),
    compiler_params=pltpu.CompilerParams(needs_layout_passes=False),
    scratch_types=[
        pltpu.VMEM_SHARED((_N, _D), jnp.float32),
        [pltpu.VMEM((_C,), jnp.int32) for _ in range(4)],
        [pltpu.VMEM((_C,), jnp.int32) for _ in range(4)],
        [pltpu.VMEM((_C, _D), jnp.float32) for _ in range(2)],
        [pltpu.VMEM((_C, _D), jnp.float32) for _ in range(2)],
        [pltpu.VMEM((_C,), jnp.float32) for _ in range(2)],
        pltpu.VMEM((_C * 16,), jnp.float32),
        [pltpu.SemaphoreType.DMA for _ in range(4)],
        [pltpu.SemaphoreType.DMA for _ in range(2)],
        [pltpu.SemaphoreType.DMA for _ in range(2)],
    ],
)
def _edge_dot(d0_hbm, d1_hbm, table_hbm, out_hbm,
              table_sh, idx0_bufs, idx1_bufs, rows0_bufs, rows1_bufs,
              out_bufs, tbuf_v, isems, gsems, osems):
    cid = lax.axis_index("c")
    sid = lax.axis_index("s")
    wid = sid * 2 + cid
    base_w = wid * _EPW

    rows_per_sub = 624
    pltpu.sync_copy(table_hbm.at[pl.ds(sid * rows_per_sub, rows_per_sub)],
                    table_sh.at[pl.ds(sid * rows_per_sub, rows_per_sub)])

    @pl.when(sid == 0)
    def _():
        rem = 16 * rows_per_sub
        pltpu.sync_copy(table_hbm.at[pl.ds(rem, _N - rem)],
                        table_sh.at[pl.ds(rem, _N - rem)])

    plsc.subcore_barrier()

    def issue_idx(t, b4):
        off = base_w + t * _C
        pltpu.async_copy(d0_hbm.at[pl.ds(off, _C)], idx0_bufs[b4], isems[b4])
        pltpu.async_copy(d1_hbm.at[pl.ds(off, _C)], idx1_bufs[b4], isems[b4])

    def wait_idx(b4):
        pltpu.make_async_copy(d0_hbm.at[pl.ds(0, _C)], idx0_bufs[b4],
                              isems[b4]).wait()
        pltpu.make_async_copy(d1_hbm.at[pl.ds(0, _C)], idx1_bufs[b4],
                              isems[b4]).wait()

    def issue_gather(b4, b2):
        pltpu.async_copy(table_sh.at[idx0_bufs[b4]], rows0_bufs[b2], gsems[b2])
        pltpu.async_copy(table_sh.at[idx1_bufs[b4]], rows1_bufs[b2], gsems[b2])

    def drain_gather(b2):
        pltpu.make_async_copy(table_hbm.at[pl.ds(0, _C)],
                              rows0_bufs[b2], gsems[b2]).wait()
        pltpu.make_async_copy(table_hbm.at[pl.ds(0, _C)],
                              rows1_bufs[b2], gsems[b2]).wait()

    def drain_out(b2):
        pltpu.make_async_copy(out_bufs[b2], out_hbm.at[pl.ds(0, _C)],
                              osems[b2]).wait()

    def compute(b2):
        r0, r1 = rows0_bufs[b2], rows1_bufs[b2]
        ob = out_bufs[b2]

        @plsc.parallel_loop(0, _C, step=1, unroll=4)
        def edge_body(e):
            p = [r0[e, pl.ds(k * 16, 16)] * r1[e, pl.ds(k * 16, 16)]
                 for k in range(_D // 16)]
            while len(p) > 1:
                p = [p[i] + p[i + 1] for i in range(0, len(p), 2)]
            tbuf_v[pl.ds(e * 16, 16)] = p[0]

        @plsc.parallel_loop(0, _G, step=1, unroll=1)
        def group_body(g):
            ids = lax.iota(jnp.int32, 16) * 16 + g * 256
            o = jnp.zeros((16,), jnp.float32)
            for l in range(16):
                o = o + plsc.load_gather(tbuf_v, [ids + l])
            ob[pl.ds(g * 16, 16)] = o

    def issue_out(t, b2):
        pltpu.async_copy(out_bufs[b2],
                         out_hbm.at[pl.ds(base_w + t * _C, _C)], osems[b2])

    for c in range(4):
        issue_idx(c, c)
    for b in range(2):
        wait_idx(b)
        issue_gather(b, b)

    def loop_body(tt, carry):
        for b in range(4):
            t = tt * 4 + b
            b2 = b % 2
            drain_gather(b2)

            @pl.when(t + 4 < _NCH)
            def _():
                issue_idx(t + 4, b)

            @pl.when(t >= 2)
            def _():
                drain_out(b2)

            compute(b2)
            issue_out(t, b2)

            @pl.when(t + 2 < _NCH)
            def _():
                wait_idx((b + 2) % 4)
                issue_gather((b + 2) % 4, b2)

        return carry

    lax.fori_loop(0, _NCH // 4, loop_body, 0)

    drain_gather(0)
    drain_out(0)
    compute(0)
    issue_out(_NCH - 1, 0)

    drain_out(0)
    drain_out(1)


def kernel(data, embedding):
    return _edge_dot(data[0], data[1], embedding)
